# Initial kernel scaffold; baseline (speedup 1.0000x reference)
import jax
import jax.numpy as jnp
from jax.experimental import pallas as pl


def kernel(hg_hat, hb_hat, dx, ghost_local_idx, a, b, lamR_raw, lamb_raw, lamd_raw, c_table):
    raise NotImplementedError("write your pallas kernel here")


# speedup vs baseline: 77.6938x; 77.6938x over previous
import functools

import jax
import jax.numpy as jnp
from jax import lax
from jax.experimental import pallas as pl
from jax.experimental.pallas import tpu as pltpu
from jax.experimental.pallas import tpu_sc as plsc

N = 3276800
NC = 2
NS = 16
NW = NC * NS
PER_W = N // NW
CHUNK = 4096
NCHUNK = PER_W // CHUNK
LANES = 16
EPS = 1e-8


def _body(hg_hbm, hb_hbm, dx_hbm, idx_hbm, consts_hbm, table_hbm,
          outg_hbm, outb_hbm,
          idx_v, hg_v, hb_v, dx_v, c_v, og_v, ob_v, consts_v, sem):
    wid = lax.axis_index("s") * NC + lax.axis_index("c")
    base = wid * PER_W

    pltpu.sync_copy(consts_hbm, consts_v)
    a16 = consts_v[0, :]
    b16 = consts_v[1, :]
    lamR = consts_v[2, :]
    lamb = consts_v[3, :]
    lamd = consts_v[4, :]

    def chunk_body(g, carry):
        off = base + g * CHUNK
        pltpu.sync_copy(idx_hbm.at[pl.ds(off, CHUNK)], idx_v)
        gather = pltpu.async_copy(table_hbm.at[idx_v], c_v, sem)
        pltpu.sync_copy(hg_hbm.at[pl.ds(off, CHUNK)], hg_v)
        pltpu.sync_copy(hb_hbm.at[pl.ds(off, CHUNK)], hb_v)
        pltpu.sync_copy(dx_hbm.at[pl.ds(off, CHUNK)], dx_v)
        gather.wait()

        def vec_body(j, carry2):
            s = j * LANES
            dxv = jnp.maximum(dx_v[pl.ds(s, LANES)], 1e-6)
            beta = b16 / (dxv + EPS)
            alpha = a16 - beta
            la = lamR * alpha
            lb = lamR * beta
            c = c_v[pl.ds(s, LANES)]
            A = lamb + la * alpha
            B = la * beta
            C = lamd + lb * beta
            rhs1 = lamb * hg_v[pl.ds(s, LANES)] + la * c
            rhs2 = lamd * hb_v[pl.ds(s, LANES)] + lb * c
            inv = 1.0 / (A * C - B * B + EPS)
            og_v[pl.ds(s, LANES)] = (C * rhs1 - B * rhs2) * inv
            ob_v[pl.ds(s, LANES)] = (A * rhs2 - B * rhs1) * inv
            return carry2

        lax.fori_loop(0, CHUNK // LANES, vec_body, 0, unroll=2)
        pltpu.sync_copy(og_v, outg_hbm.at[pl.ds(off, CHUNK)])
        pltpu.sync_copy(ob_v, outb_hbm.at[pl.ds(off, CHUNK)])
        return carry

    lax.fori_loop(0, NCHUNK, chunk_body, 0)


def kernel(hg_hat, hb_hat, dx, ghost_local_idx, a, b, lamR_raw, lamb_raw,
           lamd_raw, c_table):
    f32 = jnp.float32
    lamR = jax.nn.softplus(lamR_raw) + EPS
    lamb = jax.nn.softplus(lamb_raw) + EPS
    lamd = jax.nn.softplus(lamd_raw) + EPS
    consts = jnp.broadcast_to(
        jnp.stack([a, b, lamR, lamb, lamd]).astype(f32).reshape(5, 1), (5, 16)
    )

    run = pl.kernel(
        _body,
        out_type=(
            jax.ShapeDtypeStruct((N,), f32),
            jax.ShapeDtypeStruct((N,), f32),
        ),
        mesh=plsc.VectorSubcoreMesh(core_axis_name="c", subcore_axis_name="s"),
        scratch_types=(
            pltpu.VMEM((CHUNK,), jnp.int32),
            pltpu.VMEM((CHUNK,), f32),
            pltpu.VMEM((CHUNK,), f32),
            pltpu.VMEM((CHUNK,), f32),
            pltpu.VMEM((CHUNK,), f32),
            pltpu.VMEM((CHUNK,), f32),
            pltpu.VMEM((CHUNK,), f32),
            pltpu.VMEM((5, 16), f32),
            pltpu.SemaphoreType.DMA,
        ),
    )
    outg, outb = run(
        hg_hat.reshape(N),
        hb_hat.reshape(N),
        dx.reshape(N),
        ghost_local_idx.astype(jnp.int32),
        consts,
        c_table.reshape(-1).astype(f32),
    )
    return (outg.reshape(N, 1), outb.reshape(N, 1))


# speedup vs baseline: 94.1686x; 1.2120x over previous
import functools

import jax
import jax.numpy as jnp
from jax import lax
from jax.experimental import pallas as pl
from jax.experimental.pallas import tpu as pltpu
from jax.experimental.pallas import tpu_sc as plsc

N = 3276800
V = 1000000
NC = 2
NS = 16
NW = NC * NS
PER_W = N // NW
CHUNK = 4096
NCHUNK = PER_W // CHUNK
LANES = 16
EPS = 1e-8
STAGE_PIECE = 25000
NPIECES = V // STAGE_PIECE


def _body(hg_hbm, hb_hbm, dx_hbm, idx_hbm, consts_hbm, table_hbm,
          outg_hbm, outb_hbm,
          idx_v, hg_v, hb_v, dx_v, c_v, og_v, ob_v, consts_v, stage_v,
          tab_sh, sem):
    sid = lax.axis_index("s")
    wid = sid * NC + lax.axis_index("c")
    base = wid * PER_W

    for r in range((NPIECES + NS - 1) // NS):
        p = sid + r * NS

        @pl.when(p < NPIECES)
        def _():
            off = p * STAGE_PIECE
            pltpu.sync_copy(table_hbm.at[pl.ds(off, STAGE_PIECE)], stage_v)
            pltpu.sync_copy(stage_v, tab_sh.at[pl.ds(off, STAGE_PIECE)])

    plsc.subcore_barrier()

    pltpu.sync_copy(consts_hbm, consts_v)
    a16 = consts_v[0, :]
    b16 = consts_v[1, :]
    lamR = consts_v[2, :]
    lamb = consts_v[3, :]
    lamd = consts_v[4, :]

    def chunk_body(g, carry):
        off = base + g * CHUNK
        pltpu.sync_copy(idx_hbm.at[pl.ds(off, CHUNK)], idx_v)
        gather = pltpu.async_copy(tab_sh.at[idx_v], c_v, sem)
        pltpu.sync_copy(hg_hbm.at[pl.ds(off, CHUNK)], hg_v)
        pltpu.sync_copy(hb_hbm.at[pl.ds(off, CHUNK)], hb_v)
        pltpu.sync_copy(dx_hbm.at[pl.ds(off, CHUNK)], dx_v)
        gather.wait()

        def vec_body(j, carry2):
            s = j * LANES
            dxv = jnp.maximum(dx_v[pl.ds(s, LANES)], 1e-6)
            beta = b16 / (dxv + EPS)
            alpha = a16 - beta
            la = lamR * alpha
            lb = lamR * beta
            c = c_v[pl.ds(s, LANES)]
            A = lamb + la * alpha
            B = la * beta
            C = lamd + lb * beta
            rhs1 = lamb * hg_v[pl.ds(s, LANES)] + la * c
            rhs2 = lamd * hb_v[pl.ds(s, LANES)] + lb * c
            inv = 1.0 / (A * C - B * B + EPS)
            og_v[pl.ds(s, LANES)] = (C * rhs1 - B * rhs2) * inv
            ob_v[pl.ds(s, LANES)] = (A * rhs2 - B * rhs1) * inv
            return carry2

        lax.fori_loop(0, CHUNK // LANES, vec_body, 0, unroll=2)
        pltpu.sync_copy(og_v, outg_hbm.at[pl.ds(off, CHUNK)])
        pltpu.sync_copy(ob_v, outb_hbm.at[pl.ds(off, CHUNK)])
        return carry

    lax.fori_loop(0, NCHUNK, chunk_body, 0)


def kernel(hg_hat, hb_hat, dx, ghost_local_idx, a, b, lamR_raw, lamb_raw,
           lamd_raw, c_table):
    f32 = jnp.float32
    lamR = jax.nn.softplus(lamR_raw) + EPS
    lamb = jax.nn.softplus(lamb_raw) + EPS
    lamd = jax.nn.softplus(lamd_raw) + EPS
    consts = jnp.broadcast_to(
        jnp.stack([a, b, lamR, lamb, lamd]).astype(f32).reshape(5, 1), (5, 16)
    )

    run = pl.kernel(
        _body,
        out_type=(
            jax.ShapeDtypeStruct((N,), f32),
            jax.ShapeDtypeStruct((N,), f32),
        ),
        mesh=plsc.VectorSubcoreMesh(core_axis_name="c", subcore_axis_name="s"),
        scratch_types=(
            pltpu.VMEM((CHUNK,), jnp.int32),
            pltpu.VMEM((CHUNK,), f32),
            pltpu.VMEM((CHUNK,), f32),
            pltpu.VMEM((CHUNK,), f32),
            pltpu.VMEM((CHUNK,), f32),
            pltpu.VMEM((CHUNK,), f32),
            pltpu.VMEM((CHUNK,), f32),
            pltpu.VMEM((5, 16), f32),
            pltpu.VMEM((STAGE_PIECE,), f32),
            pltpu.VMEM_SHARED((V,), f32),
            pltpu.SemaphoreType.DMA,
        ),
    )
    outg, outb = run(
        hg_hat.reshape(N),
        hb_hat.reshape(N),
        dx.reshape(N),
        ghost_local_idx.astype(jnp.int32),
        consts,
        c_table.reshape(-1).astype(f32),
    )
    return (outg.reshape(N, 1), outb.reshape(N, 1))


# speedup vs baseline: 112.7569x; 1.1974x over previous
import jax
import jax.numpy as jnp
from jax import lax
from jax.experimental import pallas as pl
from jax.experimental.pallas import tpu as pltpu
from jax.experimental.pallas import tpu_sc as plsc

N = 3276800
V = 1000000
NC = 2
NS = 16
NW = NC * NS
PER_W = N // NW
CHUNK = 4096
NCHUNK = PER_W // CHUNK
LANES = 16
EPS = 1e-8
STAGE_PIECE = 10000
NPIECES = V // STAGE_PIECE


def _body(hg_hbm, hb_hbm, dx_hbm, idx_hbm, consts_hbm, table_hbm,
          outg_hbm, outb_hbm,
          idx_v, hg_v, hb_v, dx_v, c_v, og_v, ob_v, consts_v, stage_v,
          tab_sh, semI, semL, semG, semS):
    sid = lax.axis_index("s")
    wid = sid * NC + lax.axis_index("c")
    base = wid * PER_W

    for r in range((NPIECES + NS - 1) // NS):
        p = sid + r * NS

        @pl.when(p < NPIECES)
        def _():
            off = p * STAGE_PIECE
            pltpu.sync_copy(table_hbm.at[pl.ds(off, STAGE_PIECE)], stage_v)
            pltpu.sync_copy(stage_v, tab_sh.at[pl.ds(off, STAGE_PIECE)])

    plsc.subcore_barrier()

    pltpu.sync_copy(consts_hbm, consts_v)
    a16 = consts_v[0, :]
    b16 = consts_v[1, :]
    lamR = consts_v[2, :]
    lamb = consts_v[3, :]
    lamd = consts_v[4, :]

    loads = {}
    gathers = {}
    stores = {}

    def fire_loads(g):
        b = g % 2
        off = base + g * CHUNK
        loads[g] = (
            pltpu.async_copy(idx_hbm.at[pl.ds(off, CHUNK)], idx_v[b], semI[b]),
            pltpu.async_copy(hg_hbm.at[pl.ds(off, CHUNK)], hg_v[b], semL[b]),
            pltpu.async_copy(hb_hbm.at[pl.ds(off, CHUNK)], hb_v[b], semL[b]),
            pltpu.async_copy(dx_hbm.at[pl.ds(off, CHUNK)], dx_v[b], semL[b]),
        )

    def fire_gather(g):
        b = g % 2
        gathers[g] = pltpu.async_copy(tab_sh.at[idx_v[b]], c_v[b], semG[b])

    fire_loads(0)
    loads[0][0].wait()
    fire_gather(0)
    if NCHUNK > 1:
        fire_loads(1)

    for g in range(NCHUNK):
        b = g % 2
        off = base + g * CHUNK
        if g >= 2:
            stores[g - 2][0].wait()
            stores[g - 2][1].wait()
        loads[g][1].wait()
        loads[g][2].wait()
        loads[g][3].wait()
        gathers[g].wait()
        if g + 1 < NCHUNK:
            loads[g + 1][0].wait()
            fire_gather(g + 1)

        def vec_body(j, carry, b=b):
            s = j * LANES
            dxv = jnp.maximum(dx_v[b][pl.ds(s, LANES)], 1e-6)
            beta = b16 / (dxv + EPS)
            alpha = a16 - beta
            la = lamR * alpha
            lb = lamR * beta
            c = c_v[b][pl.ds(s, LANES)]
            A = lamb + la * alpha
            B = la * beta
            C = lamd + lb * beta
            rhs1 = lamb * hg_v[b][pl.ds(s, LANES)] + la * c
            rhs2 = lamd * hb_v[b][pl.ds(s, LANES)] + lb * c
            inv = 1.0 / (A * C - B * B + EPS)
            og_v[b][pl.ds(s, LANES)] = (C * rhs1 - B * rhs2) * inv
            ob_v[b][pl.ds(s, LANES)] = (A * rhs2 - B * rhs1) * inv
            return carry

        lax.fori_loop(0, CHUNK // LANES, vec_body, 0, unroll=2)

        stores[g] = (
            pltpu.async_copy(og_v[b], outg_hbm.at[pl.ds(off, CHUNK)], semS[b]),
            pltpu.async_copy(ob_v[b], outb_hbm.at[pl.ds(off, CHUNK)], semS[b]),
        )
        if g + 2 < NCHUNK:
            fire_loads(g + 2)

    for g in (NCHUNK - 2, NCHUNK - 1):
        if g >= 0:
            stores[g][0].wait()
            stores[g][1].wait()


def kernel(hg_hat, hb_hat, dx, ghost_local_idx, a, b, lamR_raw, lamb_raw,
           lamd_raw, c_table):
    f32 = jnp.float32
    lamR = jax.nn.softplus(lamR_raw) + EPS
    lamb = jax.nn.softplus(lamb_raw) + EPS
    lamd = jax.nn.softplus(lamd_raw) + EPS
    consts = jnp.broadcast_to(
        jnp.stack([a, b, lamR, lamb, lamd]).astype(f32).reshape(5, 1), (5, 16)
    )

    dbl = lambda spec: (spec, spec)
    run = pl.kernel(
        _body,
        out_type=(
            jax.ShapeDtypeStruct((N,), f32),
            jax.ShapeDtypeStruct((N,), f32),
        ),
        mesh=plsc.VectorSubcoreMesh(core_axis_name="c", subcore_axis_name="s"),
        scratch_types=(
            dbl(pltpu.VMEM((CHUNK,), jnp.int32)),
            dbl(pltpu.VMEM((CHUNK,), f32)),
            dbl(pltpu.VMEM((CHUNK,), f32)),
            dbl(pltpu.VMEM((CHUNK,), f32)),
            dbl(pltpu.VMEM((CHUNK,), f32)),
            dbl(pltpu.VMEM((CHUNK,), f32)),
            dbl(pltpu.VMEM((CHUNK,), f32)),
            pltpu.VMEM((5, 16), f32),
            pltpu.VMEM((STAGE_PIECE,), f32),
            pltpu.VMEM_SHARED((V,), f32),
            dbl(pltpu.SemaphoreType.DMA),
            dbl(pltpu.SemaphoreType.DMA),
            dbl(pltpu.SemaphoreType.DMA),
            dbl(pltpu.SemaphoreType.DMA),
        ),
    )
    outg, outb = run(
        hg_hat.reshape(N),
        hb_hat.reshape(N),
        dx.reshape(N),
        ghost_local_idx.astype(jnp.int32),
        consts,
        c_table.reshape(-1).astype(f32),
    )
    return (outg.reshape(N, 1), outb.reshape(N, 1))
